# baseline (speedup 1.0000x reference)
import functools

import jax
import jax.numpy as jnp
from jax import lax
from jax.experimental import pallas as pl
from jax.experimental.pallas import tpu as pltpu
from jax.experimental.pallas import tpu_sc as plsc

CB = 8192
LAT = 32
NPIX = 16384
PTILE = 256
CCHUNK = 512
NTILES = NPIX // PTILE
NCHUNKS = CB // CCHUNK


def _dist_argmin_body(x_ref, cb_ref, idx_ref, loss_ref):
    x = x_ref[...]
    xsq = jnp.sum(x * x, axis=1, keepdims=True)
    best_val = jnp.full((PTILE,), 3.4e38, jnp.float32)
    best_idx = jnp.zeros((PTILE,), jnp.int32)
    for c in range(NCHUNKS):
        cb = cb_ref[c * CCHUNK:(c + 1) * CCHUNK, :]
        csq = jnp.sum(cb * cb, axis=1)
        dot = lax.dot_general(x, cb, (((1,), (1,)), ((), ())),
                              preferred_element_type=jnp.float32)
        d = (xsq - 2.0 * dot) + csq[None, :]
        m = jnp.min(d, axis=1)
        ii = lax.broadcasted_iota(jnp.int32, d.shape, 1)
        cand = jnp.min(jnp.where(d == m[:, None], ii, jnp.int32(2 ** 30)),
                       axis=1) + c * CCHUNK
        upd = m < best_val
        best_val = jnp.where(upd, m, best_val)
        best_idx = jnp.where(upd, cand, best_idx)
    idx_ref[0, 0, :] = best_idx
    part = jnp.sum(best_val)

    @pl.when(pl.program_id(0) == 0)
    def _():
        loss_ref[0, 0] = 0.0

    loss_ref[0, 0] += part


def _dist_argmin(xt, codebook):
    return pl.pallas_call(
        _dist_argmin_body,
        grid=(NTILES,),
        in_specs=[
            pl.BlockSpec((PTILE, LAT), lambda i: (i, 0)),
            pl.BlockSpec((CB, LAT), lambda i: (0, 0)),
        ],
        out_specs=[
            pl.BlockSpec((1, 1, PTILE), lambda i: (i, 0, 0)),
            pl.BlockSpec(memory_space=pltpu.SMEM),
        ],
        out_shape=[
            jax.ShapeDtypeStruct((NTILES, 1, PTILE), jnp.int32),
            jax.ShapeDtypeStruct((1, 1), jnp.float32),
        ],
        compiler_params=pltpu.CompilerParams(
            dimension_semantics=("arbitrary",)),
    )(xt, codebook)


NCORES = 2
NSUB = 16
NW = NCORES * NSUB
BPW = NPIX // NW


NB = 16
PIX = 1024


@functools.cache
def _sc_st_kernel():
    @functools.partial(
        pl.kernel,
        mesh=plsc.VectorSubcoreMesh(core_axis_name="c", subcore_axis_name="s"),
        out_type=jax.ShapeDtypeStruct((NPIX * LAT,), jnp.float32),
        scratch_types=[
            pltpu.VMEM((NPIX,), jnp.int32),
            pltpu.VMEM((NPIX,), jnp.float32),
            pltpu.VMEM((NPIX,), jnp.float32),
            pltpu.VMEM((CB,), jnp.float32),
            pltpu.SemaphoreType.DMA,
        ],
        compiler_params=pltpu.CompilerParams(needs_layout_passes=False),
    )
    def _sc_st(cbt_hbm, idx_hbm, x_hbm, out_hbm, idx_v, x_v, o_v, cbt_v, sem):
        w = lax.axis_index("s") * NCORES + lax.axis_index("c")
        copies = [
            pltpu.make_async_copy(cbt_hbm.at[pl.ds(w * CB, CB)], cbt_v, sem),
            pltpu.make_async_copy(idx_hbm, idx_v, sem),
        ]
        for b in range(NB):
            src = x_hbm.at[pl.ds((b * LAT + w) * PIX, PIX)]
            copies.append(pltpu.make_async_copy(
                src, x_v.at[pl.ds(b * PIX, PIX)], sem))
        for c in copies:
            c.start()
        for c in copies:
            c.wait()

        def body(g, carry):
            o = pl.multiple_of(g * 16, 16)
            iv = idx_v[pl.ds(o, 16)]
            q = plsc.load_gather(cbt_v, [iv])
            xv = x_v[pl.ds(o, 16)]
            o_v[pl.ds(o, 16)] = xv + (q - xv)
            return carry

        lax.fori_loop(0, NPIX // 16, body, 0)

        outs = []
        for b in range(NB):
            dst = out_hbm.at[pl.ds((b * LAT + w) * PIX, PIX)]
            outs.append(pltpu.make_async_copy(
                o_v.at[pl.ds(b * PIX, PIX)], dst, sem))
        for c in outs:
            c.start()
        for c in outs:
            c.wait()

    return _sc_st


def kernel(inputs, codebook):
    xt = jnp.transpose(inputs.reshape(NB, LAT, PIX), (0, 2, 1))
    xt = xt.reshape(NPIX, LAT)
    idx3, loss = _dist_argmin(xt, codebook)
    idx = idx3.reshape(NPIX)
    cbt = jnp.transpose(codebook).reshape(CB * LAT)
    x_flat = inputs.reshape(NPIX * LAT)
    st_flat = _sc_st_kernel()(cbt, idx, x_flat)
    quantized_out = st_flat.reshape(NB, LAT, 32, 32)
    m = loss[0, 0] / jnp.float32(NPIX * LAT)
    embedding_loss = m
    commitment_loss = m
    vq_loss = commitment_loss * 0.25 + embedding_loss
    return quantized_out, vq_loss, embedding_loss, commitment_loss
